# Initial kernel scaffold; baseline (speedup 1.0000x reference)
import jax
import jax.numpy as jnp
from jax.experimental import pallas as pl


def kernel(x, Wg, bg, Wn, bn, W1, b1, W2, b2):
    raise NotImplementedError("write your pallas kernel here")


# speedup vs baseline: 2.4542x; 2.4542x over previous
import functools

import jax
import jax.numpy as jnp
from jax import lax
from jax.experimental import pallas as pl
from jax.experimental.pallas import tpu as pltpu
import jax.experimental.pallas.tpu_sc as plsc

_E = 8
_K = 2
_N = 4096
_C = 1024
_H = 4096
_CAP = _N * _K // _E
_HBLK = 512
_NW = 32
_TPW = _N // _NW
_CHUNK = 64


@functools.cache
def _make_dispatch():
    mesh = plsc.VectorSubcoreMesh(core_axis_name="c", subcore_axis_name="s")

    @functools.partial(
        pl.kernel,
        out_type=jax.ShapeDtypeStruct((_E * _CAP + 1024, _C), jnp.float32),
        mesh=mesh,
        scratch_types=[
            pltpu.VMEM((_CHUNK,), jnp.int32),
            pltpu.VMEM((_CHUNK, _C), jnp.float32),
            pltpu.SemaphoreType.DMA,
        ],
    )
    def _dispatch(x_hbm, slot_hbm, xbuf_hbm, idx_v, rows_v, sem):
        wid = lax.axis_index("s") * 2 + lax.axis_index("c")
        base = wid * _TPW
        for c in range(_TPW // _CHUNK):
            b = base + c * _CHUNK
            pltpu.sync_copy(slot_hbm.at[pl.ds(b, _CHUNK)], idx_v)
            pltpu.sync_copy(x_hbm.at[pl.ds(b, _CHUNK)], rows_v)
            pltpu.async_copy(rows_v, xbuf_hbm.at[idx_v], sem).wait()

    return _dispatch


_RBLK = 256


def _ffn_body(cnt_ref, x_ref, w1_ref, b1_ref, w2_ref, b2_ref, out_ref):
    e = pl.program_id(0)
    h_idx = pl.program_id(1)
    cnt = cnt_ref[e]
    for r in range(_CAP // _RBLK):
        active = r * _RBLK < cnt
        rows = pl.ds(r * _RBLK, _RBLK)

        @pl.when(jnp.logical_and(h_idx == 0, jnp.logical_not(active)))
        def _():
            out_ref[rows, :] = jnp.zeros((_RBLK, _C), jnp.float32)

        @pl.when(active)
        def _():
            xb = x_ref[rows, :]
            h = jnp.maximum(
                jnp.dot(xb, w1_ref[0], preferred_element_type=jnp.float32)
                + b1_ref[0, 0, :][None, :], 0.0)
            h = h * h
            part = jnp.dot(h, w2_ref[0], preferred_element_type=jnp.float32)

            @pl.when(h_idx == 0)
            def _():
                out_ref[rows, :] = part + b2_ref[0, 0, :][None, :]

            @pl.when(h_idx != 0)
            def _():
                out_ref[rows, :] += part


def _ffn(counts, xbuf, W1, b1r, W2, b2r):
    return pl.pallas_call(
        _ffn_body,
        grid_spec=pltpu.PrefetchScalarGridSpec(
            num_scalar_prefetch=1,
            grid=(_E, _H // _HBLK),
            in_specs=[
                pl.BlockSpec((_CAP, _C), lambda e, h, c: (e, 0)),
                pl.BlockSpec((1, _C, _HBLK), lambda e, h, c: (e, 0, h)),
                pl.BlockSpec((1, 1, _HBLK), lambda e, h, c: (e, 0, h)),
                pl.BlockSpec((1, _HBLK, _C), lambda e, h, c: (e, h, 0)),
                pl.BlockSpec((1, 1, _C), lambda e, h, c: (e, 0, 0)),
            ],
            out_specs=pl.BlockSpec((_CAP, _C), lambda e, h, c: (e, 0)),
        ),
        out_shape=jax.ShapeDtypeStruct((_E * _CAP, _C), jnp.float32),
    )(counts, xbuf, W1, b1r, W2, b2r)


@functools.cache
def _make_combine():
    mesh = plsc.VectorSubcoreMesh(core_axis_name="c", subcore_axis_name="s")

    @functools.partial(
        pl.kernel,
        out_type=jax.ShapeDtypeStruct((_N, _C), jnp.float32),
        mesh=mesh,
        scratch_types=[
            pltpu.VMEM((_CHUNK,), jnp.int32),
            pltpu.VMEM((_CHUNK, 16), jnp.float32),
            pltpu.VMEM((_CHUNK, _C), jnp.float32),
            pltpu.SemaphoreType.DMA,
        ],
    )
    def _combine(ybuf_hbm, slot_hbm, gate_hbm, out_hbm, idx_v, g_v, rows_v, sem):
        wid = lax.axis_index("s") * 2 + lax.axis_index("c")
        base = wid * _TPW
        for c in range(_TPW // _CHUNK):
            b = base + c * _CHUNK
            pltpu.sync_copy(slot_hbm.at[pl.ds(b, _CHUNK)], idx_v)
            pltpu.sync_copy(gate_hbm.at[pl.ds(b, _CHUNK)], g_v)
            pltpu.async_copy(ybuf_hbm.at[idx_v], rows_v, sem).wait()

            def body(j, carry):
                gv = g_v[j, :]
                for v in range(_C // 16):
                    rows_v[j, pl.ds(v * 16, 16)] = rows_v[j, pl.ds(v * 16, 16)] * gv
                return carry

            lax.fori_loop(0, _CHUNK, body, 0)
            pltpu.sync_copy(rows_v, out_hbm.at[pl.ds(b, _CHUNK)])

    return _combine


def kernel(x, Wg, bg, Wn, bn, W1, b1, W2, b2):
    B, T, C = x.shape
    E = Wg.shape[1]
    logits = x @ Wg + bg
    noise_logits = x @ Wn + bn
    noise = jax.random.normal(jax.random.key(42), logits.shape,
                              dtype=logits.dtype) * jax.nn.softplus(noise_logits)
    noisy = logits + noise
    topv, topi = jax.lax.top_k(noisy, _K)
    keep = (jax.nn.one_hot(topi, E, dtype=jnp.float32).sum(axis=-2) > 0)
    sparse_logits = jnp.where(keep, noisy, -jnp.inf)
    gating = jax.nn.softmax(sparse_logits, axis=-1)

    flat_x = x.reshape(-1, C)
    flat_g = gating.reshape(-1, E)
    top1 = jnp.argmax(flat_g, axis=-1)
    gate = jnp.take_along_axis(flat_g, top1[:, None], axis=-1)[:, 0]
    onehot = (top1[:, None] == jnp.arange(E)[None, :]).astype(jnp.int32)
    cum = jnp.cumsum(onehot, axis=0)
    rank = jnp.take_along_axis(cum, top1[:, None], axis=-1)[:, 0] - 1
    valid = rank < _CAP
    slot = top1.astype(jnp.int32) * _CAP + rank.astype(jnp.int32)
    slot_sc = jnp.where(valid, slot, _E * _CAP).astype(jnp.int32)
    slot_g = jnp.where(valid, slot, 0).astype(jnp.int32)
    gate_m = jnp.where(valid, gate, 0.0).astype(jnp.float32)
    gate_b = jnp.broadcast_to(gate_m[:, None], (_N, 16))

    counts = jnp.minimum(cum[-1], _CAP).astype(jnp.int32)
    xbuf = _make_dispatch()(flat_x, slot_sc)
    ybuf = _ffn(counts, xbuf, W1, b1.reshape(E, 1, _H), W2, b2.reshape(E, 1, _C))
    out = _make_combine()(ybuf, slot_g, gate_b)
    return out.reshape(B, T, C)


# speedup vs baseline: 2.4555x; 1.0005x over previous
import functools

import jax
import jax.numpy as jnp
from jax import lax
from jax.experimental import pallas as pl
from jax.experimental.pallas import tpu as pltpu
import jax.experimental.pallas.tpu_sc as plsc

_E = 8
_K = 2
_N = 4096
_C = 1024
_H = 4096
_CAP = _N * _K // _E
_HBLK = 512
_NW = 32
_TPW = _N // _NW
_CHUNK = 64


@functools.cache
def _make_dispatch():
    mesh = plsc.VectorSubcoreMesh(core_axis_name="c", subcore_axis_name="s")

    @functools.partial(
        pl.kernel,
        out_type=jax.ShapeDtypeStruct((_E * _CAP + 1024, _C), jnp.float32),
        mesh=mesh,
        scratch_types=[
            pltpu.VMEM((_CHUNK,), jnp.int32),
            pltpu.VMEM((_CHUNK, _C), jnp.float32),
            pltpu.SemaphoreType.DMA,
        ],
    )
    def _dispatch(x_hbm, slot_hbm, xbuf_hbm, idx_v, rows_v, sem):
        wid = lax.axis_index("s") * 2 + lax.axis_index("c")
        base = wid * _TPW
        for c in range(_TPW // _CHUNK):
            b = base + c * _CHUNK
            pltpu.sync_copy(slot_hbm.at[pl.ds(b, _CHUNK)], idx_v)
            pltpu.sync_copy(x_hbm.at[pl.ds(b, _CHUNK)], rows_v)
            pltpu.async_copy(rows_v, xbuf_hbm.at[idx_v], sem).wait()

    return _dispatch


_RBLK = 256


def _ffn_body(cnt_ref, x_ref, w1_ref, b1_ref, w2_ref, b2_ref, out_ref):
    e = pl.program_id(0)
    h_idx = pl.program_id(1)
    cnt = cnt_ref[e]
    for r in range(_CAP // _RBLK):
        active = r * _RBLK < cnt
        rows = pl.ds(r * _RBLK, _RBLK)

        @pl.when(jnp.logical_and(h_idx == 0, jnp.logical_not(active)))
        def _():
            out_ref[rows, :] = jnp.zeros((_RBLK, _C), jnp.float32)

        @pl.when(active)
        def _():
            xb = x_ref[rows, :].astype(jnp.bfloat16)
            w1b = w1_ref[0].astype(jnp.bfloat16)
            h = jnp.maximum(
                jnp.dot(xb, w1b, preferred_element_type=jnp.float32)
                + b1_ref[0, 0, :][None, :], 0.0)
            h = (h * h).astype(jnp.bfloat16)
            w2b = w2_ref[0].astype(jnp.bfloat16)
            part = jnp.dot(h, w2b, preferred_element_type=jnp.float32)

            @pl.when(h_idx == 0)
            def _():
                out_ref[rows, :] = part + b2_ref[0, 0, :][None, :]

            @pl.when(h_idx != 0)
            def _():
                out_ref[rows, :] += part


def _ffn(counts, xbuf, W1, b1r, W2, b2r):
    return pl.pallas_call(
        _ffn_body,
        grid_spec=pltpu.PrefetchScalarGridSpec(
            num_scalar_prefetch=1,
            grid=(_E, _H // _HBLK),
            in_specs=[
                pl.BlockSpec((_CAP, _C), lambda e, h, c: (e, 0)),
                pl.BlockSpec((1, _C, _HBLK), lambda e, h, c: (e, 0, h)),
                pl.BlockSpec((1, 1, _HBLK), lambda e, h, c: (e, 0, h)),
                pl.BlockSpec((1, _HBLK, _C), lambda e, h, c: (e, h, 0)),
                pl.BlockSpec((1, 1, _C), lambda e, h, c: (e, 0, 0)),
            ],
            out_specs=pl.BlockSpec((_CAP, _C), lambda e, h, c: (e, 0)),
        ),
        out_shape=jax.ShapeDtypeStruct((_E * _CAP, _C), jnp.float32),
    )(counts, xbuf, W1, b1r, W2, b2r)


@functools.cache
def _make_combine():
    mesh = plsc.VectorSubcoreMesh(core_axis_name="c", subcore_axis_name="s")

    @functools.partial(
        pl.kernel,
        out_type=jax.ShapeDtypeStruct((_N, _C), jnp.float32),
        mesh=mesh,
        scratch_types=[
            pltpu.VMEM((_CHUNK,), jnp.int32),
            pltpu.VMEM((_CHUNK, 16), jnp.float32),
            pltpu.VMEM((_CHUNK, _C), jnp.float32),
            pltpu.SemaphoreType.DMA,
        ],
    )
    def _combine(ybuf_hbm, slot_hbm, gate_hbm, out_hbm, idx_v, g_v, rows_v, sem):
        wid = lax.axis_index("s") * 2 + lax.axis_index("c")
        base = wid * _TPW
        for c in range(_TPW // _CHUNK):
            b = base + c * _CHUNK
            pltpu.sync_copy(slot_hbm.at[pl.ds(b, _CHUNK)], idx_v)
            pltpu.sync_copy(gate_hbm.at[pl.ds(b, _CHUNK)], g_v)
            pltpu.async_copy(ybuf_hbm.at[idx_v], rows_v, sem).wait()

            def body(j, carry):
                gv = g_v[j, :]
                for v in range(_C // 16):
                    rows_v[j, pl.ds(v * 16, 16)] = rows_v[j, pl.ds(v * 16, 16)] * gv
                return carry

            lax.fori_loop(0, _CHUNK, body, 0)
            pltpu.sync_copy(rows_v, out_hbm.at[pl.ds(b, _CHUNK)])

    return _combine


def kernel(x, Wg, bg, Wn, bn, W1, b1, W2, b2):
    B, T, C = x.shape
    E = Wg.shape[1]
    logits = x @ Wg + bg
    noise_logits = x @ Wn + bn
    noise = jax.random.normal(jax.random.key(42), logits.shape,
                              dtype=logits.dtype) * jax.nn.softplus(noise_logits)
    noisy = logits + noise
    topv, topi = jax.lax.top_k(noisy, _K)
    keep = (jax.nn.one_hot(topi, E, dtype=jnp.float32).sum(axis=-2) > 0)
    sparse_logits = jnp.where(keep, noisy, -jnp.inf)
    gating = jax.nn.softmax(sparse_logits, axis=-1)

    flat_x = x.reshape(-1, C)
    flat_g = gating.reshape(-1, E)
    top1 = jnp.argmax(flat_g, axis=-1)
    gate = jnp.take_along_axis(flat_g, top1[:, None], axis=-1)[:, 0]
    onehot = (top1[:, None] == jnp.arange(E)[None, :]).astype(jnp.int32)
    cum = jnp.cumsum(onehot, axis=0)
    rank = jnp.take_along_axis(cum, top1[:, None], axis=-1)[:, 0] - 1
    valid = rank < _CAP
    slot = top1.astype(jnp.int32) * _CAP + rank.astype(jnp.int32)
    slot_sc = jnp.where(valid, slot, _E * _CAP).astype(jnp.int32)
    slot_g = jnp.where(valid, slot, 0).astype(jnp.int32)
    gate_m = jnp.where(valid, gate, 0.0).astype(jnp.float32)
    gate_b = jnp.broadcast_to(gate_m[:, None], (_N, 16))

    counts = jnp.minimum(cum[-1], _CAP).astype(jnp.int32)
    xbuf = _make_dispatch()(flat_x, slot_sc)
    ybuf = _ffn(counts, xbuf, W1, b1.reshape(E, 1, _H), W2, b2.reshape(E, 1, _C))
    out = _make_combine()(ybuf, slot_g, gate_b)
    return out.reshape(B, T, C)


# speedup vs baseline: 2.8985x; 1.1804x over previous
import functools

import jax
import jax.numpy as jnp
from jax import lax
from jax.experimental import pallas as pl
from jax.experimental.pallas import tpu as pltpu
import jax.experimental.pallas.tpu_sc as plsc

_E = 8
_K = 2
_N = 4096
_C = 1024
_H = 4096
_CAP = _N * _K // _E
_HBLK = 1024
_NW = 32
_TPW = _N // _NW
_CHUNK = 64


@functools.cache
def _make_dispatch():
    mesh = plsc.VectorSubcoreMesh(core_axis_name="c", subcore_axis_name="s")

    @functools.partial(
        pl.kernel,
        out_type=jax.ShapeDtypeStruct((_E * _CAP + 1024, _C), jnp.float32),
        mesh=mesh,
        scratch_types=[
            pltpu.VMEM((_CHUNK,), jnp.int32),
            pltpu.VMEM((_CHUNK, _C), jnp.float32),
            pltpu.SemaphoreType.DMA,
        ],
    )
    def _dispatch(x_hbm, slot_hbm, xbuf_hbm, idx_v, rows_v, sem):
        wid = lax.axis_index("s") * 2 + lax.axis_index("c")
        base = wid * _TPW
        for c in range(_TPW // _CHUNK):
            b = base + c * _CHUNK
            pltpu.sync_copy(slot_hbm.at[pl.ds(b, _CHUNK)], idx_v)
            pltpu.sync_copy(x_hbm.at[pl.ds(b, _CHUNK)], rows_v)
            pltpu.async_copy(rows_v, xbuf_hbm.at[idx_v], sem).wait()

    return _dispatch


_RBLK = 256


def _ffn_body(cnt_ref, x_ref, w1_ref, b1_ref, w2_ref, b2_ref, out_ref):
    e = pl.program_id(0)
    h_idx = pl.program_id(1)
    cnt = cnt_ref[e]
    for r in range(_CAP // _RBLK):
        active = r * _RBLK < cnt
        rows = pl.ds(r * _RBLK, _RBLK)

        if r == 0:
            @pl.when(jnp.logical_and(
                jnp.logical_and(h_idx == 0, jnp.logical_not(active)), e == 0))
            def _():
                out_ref[rows, :] = jnp.zeros((_RBLK, _C), jnp.float32)

        @pl.when(active)
        def _():
            xb = x_ref[rows, :].astype(jnp.bfloat16)
            w1b = w1_ref[0].astype(jnp.bfloat16)
            h = jnp.maximum(
                jnp.dot(xb, w1b, preferred_element_type=jnp.float32)
                + b1_ref[0, 0, :][None, :], 0.0)
            h = (h * h).astype(jnp.bfloat16)
            w2b = w2_ref[0].astype(jnp.bfloat16)
            part = jnp.dot(h, w2b, preferred_element_type=jnp.float32)

            @pl.when(h_idx == 0)
            def _():
                out_ref[rows, :] = part + b2_ref[0, 0, :][None, :]

            @pl.when(h_idx != 0)
            def _():
                out_ref[rows, :] += part


def _ffn(counts, xbuf, W1, b1r, W2, b2r):
    return pl.pallas_call(
        _ffn_body,
        grid_spec=pltpu.PrefetchScalarGridSpec(
            num_scalar_prefetch=1,
            grid=(_E, _H // _HBLK),
            in_specs=[
                pl.BlockSpec((_CAP, _C), lambda e, h, c: (e, 0)),
                pl.BlockSpec((1, _C, _HBLK), lambda e, h, c: (e, 0, h)),
                pl.BlockSpec((1, 1, _HBLK), lambda e, h, c: (e, 0, h)),
                pl.BlockSpec((1, _HBLK, _C), lambda e, h, c: (e, h, 0)),
                pl.BlockSpec((1, 1, _C), lambda e, h, c: (e, 0, 0)),
            ],
            out_specs=pl.BlockSpec((_CAP, _C), lambda e, h, c: (e, 0)),
        ),
        out_shape=jax.ShapeDtypeStruct((_E * _CAP, _C), jnp.float32),
    )(counts, xbuf, W1, b1r, W2, b2r)


@functools.cache
def _make_combine():
    mesh = plsc.VectorSubcoreMesh(core_axis_name="c", subcore_axis_name="s")

    @functools.partial(
        pl.kernel,
        out_type=jax.ShapeDtypeStruct((_N, _C), jnp.float32),
        mesh=mesh,
        scratch_types=[
            pltpu.VMEM((_CHUNK,), jnp.int32),
            pltpu.VMEM((_CHUNK, 16), jnp.float32),
            pltpu.VMEM((_CHUNK, _C), jnp.float32),
            pltpu.SemaphoreType.DMA,
        ],
    )
    def _combine(ybuf_hbm, slot_hbm, gate_hbm, out_hbm, idx_v, g_v, rows_v, sem):
        wid = lax.axis_index("s") * 2 + lax.axis_index("c")
        base = wid * _TPW
        for c in range(_TPW // _CHUNK):
            b = base + c * _CHUNK
            pltpu.sync_copy(slot_hbm.at[pl.ds(b, _CHUNK)], idx_v)
            pltpu.sync_copy(gate_hbm.at[pl.ds(b, _CHUNK)], g_v)
            pltpu.async_copy(ybuf_hbm.at[idx_v], rows_v, sem).wait()

            def body(j, carry):
                gv = g_v[j, :]
                for v in range(_C // 16):
                    rows_v[j, pl.ds(v * 16, 16)] = rows_v[j, pl.ds(v * 16, 16)] * gv
                return carry

            lax.fori_loop(0, _CHUNK, body, 0)
            pltpu.sync_copy(rows_v, out_hbm.at[pl.ds(b, _CHUNK)])

    return _combine


def kernel(x, Wg, bg, Wn, bn, W1, b1, W2, b2):
    B, T, C = x.shape
    E = Wg.shape[1]
    logits = x @ Wg + bg
    noise_logits = x @ Wn + bn
    noise = jax.random.normal(jax.random.key(42), logits.shape,
                              dtype=logits.dtype) * jax.nn.softplus(noise_logits)
    noisy = logits + noise
    topv, topi = jax.lax.top_k(noisy, _K)
    keep = (jax.nn.one_hot(topi, E, dtype=jnp.float32).sum(axis=-2) > 0)
    sparse_logits = jnp.where(keep, noisy, -jnp.inf)
    gating = jax.nn.softmax(sparse_logits, axis=-1)

    flat_x = x.reshape(-1, C)
    flat_g = gating.reshape(-1, E)
    top1 = jnp.argmax(flat_g, axis=-1)
    gate = jnp.take_along_axis(flat_g, top1[:, None], axis=-1)[:, 0]
    onehot = (top1[:, None] == jnp.arange(E)[None, :]).astype(jnp.int32)
    cum = jnp.cumsum(onehot, axis=0)
    rank = jnp.take_along_axis(cum, top1[:, None], axis=-1)[:, 0] - 1
    valid = rank < _CAP
    slot = top1.astype(jnp.int32) * _CAP + rank.astype(jnp.int32)
    slot_sc = jnp.where(valid, slot, _E * _CAP).astype(jnp.int32)
    slot_g = jnp.where(valid, slot, 0).astype(jnp.int32)
    gate_m = jnp.where(valid, gate, 0.0).astype(jnp.float32)
    gate_b = jnp.broadcast_to(gate_m[:, None], (_N, 16))

    counts = jnp.minimum(cum[-1], _CAP).astype(jnp.int32)
    xbuf = _make_dispatch()(flat_x, slot_sc)
    ybuf = _ffn(counts, xbuf, W1, b1.reshape(E, 1, _H), W2, b2.reshape(E, 1, _C))
    out = _make_combine()(ybuf, slot_g, gate_b)
    return out.reshape(B, T, C)
